# baseline (speedup 1.0000x reference)
import jax
import jax.numpy as jnp
from jax.experimental import pallas as pl

_BS = 2048


def _add_kernel(seq_l_ref, seq_r_ref, pe_ref, out_ref):
    h = seq_l_ref.shape[-1]
    out_ref[:, :, :h] = seq_l_ref[...] + pe_ref[:, :h][None]
    out_ref[:, :, h:] = seq_r_ref[...] + pe_ref[:, h:][None]


def kernel(sequence, pe):
    b, s, d = sequence.shape
    ns = s // _BS
    h = d // 2
    return pl.pallas_call(
        _add_kernel,
        grid=(ns, b),
        in_specs=[
            pl.BlockSpec((1, _BS, h), lambda i, j: (j, i, 0)),
            pl.BlockSpec((1, _BS, h), lambda i, j: (j, i, 1)),
            pl.BlockSpec((_BS, d), lambda i, j: (i, 0)),
        ],
        out_specs=pl.BlockSpec((1, _BS, d), lambda i, j: (j, i, 0)),
        out_shape=jax.ShapeDtypeStruct((b, s, d), sequence.dtype),
    )(sequence, sequence, pe)
